# baseline (speedup 1.0000x reference)
import functools

import jax
import jax.numpy as jnp
from jax import lax
from jax.experimental import pallas as pl
from jax.experimental.pallas import tpu as pltpu
from jax.experimental.pallas import tpu_sc as plsc

N_TOKENS = 32768
HIDDEN = 4096
N_EXPERTS = 64
TOP_K = 8

BT = 512

NSLICE = 8
SLICE = N_TOKENS // NSLICE

NC = 2
NS = 16
LANES = 16
NW = NC * NS
TPW = SLICE // NW
CT = min(256, TPW)
NCHUNK = TPW // CT
NGROUP = CT // LANES


def _gate_tc_body(h_ref, w_ref, probs_ref):
    logits = lax.dot_general(
        w_ref[...], h_ref[...],
        dimension_numbers=(((1,), (1,)), ((), ())),
        preferred_element_type=jnp.float32,
    )
    m = jnp.max(logits, axis=0, keepdims=True)
    e = jnp.exp(logits - m)
    probs_ref[...] = e / jnp.sum(e, axis=0, keepdims=True)


def _probs_transposed(h, weight, sl):
    blk0 = sl * (SLICE // BT)
    return pl.pallas_call(
        _gate_tc_body,
        grid=(SLICE // BT,),
        in_specs=[
            pl.BlockSpec((BT, HIDDEN), lambda i: (blk0 + i, 0)),
            pl.BlockSpec((N_EXPERTS, HIDDEN), lambda i: (0, 0)),
        ],
        out_specs=pl.BlockSpec((N_EXPERTS, BT), lambda i: (0, i)),
        out_shape=jax.ShapeDtypeStruct((N_EXPERTS, SLICE), jnp.float32),
    )(h, weight)


def _tree_argmax(vals):
    vals = list(vals)
    idxs = list(range(len(vals)))
    while len(vals) > 1:
        nv, ni = [], []
        for j in range(0, len(vals) - 1, 2):
            va, vb = vals[j], vals[j + 1]
            ia, ib = idxs[j], idxs[j + 1]
            cond = va >= vb
            nv.append(jnp.where(cond, va, vb))
            if isinstance(ia, int) and isinstance(ib, int):
                ni.append(jnp.where(cond, jnp.int32(ia), jnp.int32(ib)))
            else:
                ni.append(jnp.where(cond, ia, ib))
        if len(vals) % 2:
            nv.append(vals[-1])
            ni.append(idxs[-1])
        vals, idxs = nv, ni
    top_i = idxs[0]
    if isinstance(top_i, int):
        top_i = jnp.full((LANES,), top_i, jnp.int32)
    return vals[0], top_i


def _sc_topk_body(probs_hbm, outw_hbm, outi_hbm, buf, outw_v, outi_v):
    wid = lax.axis_index("s") * NC + lax.axis_index("c")
    base = wid * TPW

    def chunk_body(ci, carry):
        tok0 = base + ci * CT
        pltpu.sync_copy(probs_hbm.at[:, pl.ds(tok0, CT)], buf)

        def group_body(g, carry2):
            goff = g * LANES
            s = [buf[e, pl.ds(goff, LANES)] for e in range(N_EXPERTS)]
            for k in range(TOP_K):
                v, a = _tree_argmax(s)
                outw_v[k, pl.ds(goff, LANES)] = v
                outi_v[k, pl.ds(goff, LANES)] = a
                if k + 1 < TOP_K:
                    s = [jnp.where(a == e, jnp.float32(-1.0), s[e])
                         for e in range(N_EXPERTS)]
            return carry2

        lax.fori_loop(0, NGROUP, group_body, 0)
        pltpu.sync_copy(outw_v, outw_hbm.at[:, pl.ds(tok0, CT)])
        pltpu.sync_copy(outi_v, outi_hbm.at[:, pl.ds(tok0, CT)])
        return carry

    lax.fori_loop(0, NCHUNK, chunk_body, 0)


@functools.cache
def _sc_topk():
    return pl.kernel(
        _sc_topk_body,
        out_type=(
            jax.ShapeDtypeStruct((TOP_K, SLICE), jnp.float32),
            jax.ShapeDtypeStruct((TOP_K, SLICE), jnp.int32),
        ),
        mesh=plsc.VectorSubcoreMesh(
            core_axis_name="c", subcore_axis_name="s",
            num_cores=NC, num_subcores=NS,
        ),
        scratch_types=[
            pltpu.VMEM((N_EXPERTS, CT), jnp.float32),
            pltpu.VMEM((TOP_K, CT), jnp.float32),
            pltpu.VMEM((TOP_K, CT), jnp.int32),
        ],
        compiler_params=pltpu.CompilerParams(use_tc_tiling_on_sc=False),
    )


def kernel(h, weight):
    sc = _sc_topk()
    ws, idxs = [], []
    for sl in range(NSLICE):
        probs_t = _probs_transposed(h, weight, sl)
        topk_w, topk_i = sc(probs_t)
        ws.append(topk_w.T)
        idxs.append(topk_i.T)
    return jnp.concatenate(ws, axis=0), jnp.concatenate(idxs, axis=0)


# speedup vs baseline: 1.1057x; 1.1057x over previous
import functools

import jax
import jax.numpy as jnp
from jax import lax
from jax.experimental import pallas as pl
from jax.experimental.pallas import tpu as pltpu
from jax.experimental.pallas import tpu_sc as plsc

N_TOKENS = 32768
HIDDEN = 4096
N_EXPERTS = 64
TOP_K = 8

BT = 512

SLICES = (12288, 12288, 6144, 2048)

NC = 2
NS = 16
LANES = 16
NW = NC * NS


def _gate_tc_body(h_ref, w_ref, probs_ref):
    logits = lax.dot_general(
        w_ref[...], h_ref[...],
        dimension_numbers=(((1,), (1,)), ((), ())),
        preferred_element_type=jnp.float32,
    )
    m = jnp.max(logits, axis=0, keepdims=True)
    e = jnp.exp(logits - m)
    probs_ref[...] = e / jnp.sum(e, axis=0, keepdims=True)


def _probs_transposed(h, weight, tok0, ntok):
    blk0 = tok0 // BT
    return pl.pallas_call(
        _gate_tc_body,
        grid=(ntok // BT,),
        in_specs=[
            pl.BlockSpec((BT, HIDDEN), lambda i: (blk0 + i, 0)),
            pl.BlockSpec((N_EXPERTS, HIDDEN), lambda i: (0, 0)),
        ],
        out_specs=pl.BlockSpec((N_EXPERTS, BT), lambda i: (0, i)),
        out_shape=jax.ShapeDtypeStruct((N_EXPERTS, ntok), jnp.float32),
    )(h, weight)


def _tree_argmax(vals):
    vals = list(vals)
    idxs = list(range(len(vals)))
    while len(vals) > 1:
        nv, ni = [], []
        for j in range(0, len(vals) - 1, 2):
            va, vb = vals[j], vals[j + 1]
            ia, ib = idxs[j], idxs[j + 1]
            cond = va >= vb
            nv.append(jnp.where(cond, va, vb))
            if isinstance(ia, int) and isinstance(ib, int):
                ni.append(jnp.where(cond, jnp.int32(ia), jnp.int32(ib)))
            else:
                ni.append(jnp.where(cond, ia, ib))
        if len(vals) % 2:
            nv.append(vals[-1])
            ni.append(idxs[-1])
        vals, idxs = nv, ni
    top_i = idxs[0]
    if isinstance(top_i, int):
        top_i = jnp.full((LANES,), top_i, jnp.int32)
    return vals[0], top_i


def _make_sc_topk_body(tpw, ct, nchunk, ngroup):
    def body(probs_hbm, outw_hbm, outi_hbm, buf, outw_v, outi_v):
        wid = lax.axis_index("s") * NC + lax.axis_index("c")
        base = wid * tpw

        def chunk_body(ci, carry):
            tok0 = base + ci * ct
            pltpu.sync_copy(probs_hbm.at[:, pl.ds(tok0, ct)], buf)

            def group_body(g, carry2):
                goff = g * LANES
                s = [buf[e, pl.ds(goff, LANES)] for e in range(N_EXPERTS)]
                for k in range(TOP_K):
                    v, a = _tree_argmax(s)
                    outw_v[k, pl.ds(goff, LANES)] = v
                    outi_v[k, pl.ds(goff, LANES)] = a
                    if k + 1 < TOP_K:
                        s = [jnp.where(a == e, jnp.float32(-1.0), s[e])
                             for e in range(N_EXPERTS)]
                return carry2

            lax.fori_loop(0, ngroup, group_body, 0)
            pltpu.sync_copy(outw_v, outw_hbm.at[:, pl.ds(tok0, ct)])
            pltpu.sync_copy(outi_v, outi_hbm.at[:, pl.ds(tok0, ct)])
            return carry

        lax.fori_loop(0, nchunk, chunk_body, 0)

    return body


@functools.cache
def _sc_topk(ntok):
    tpw = ntok // NW
    ct = min(256, tpw)
    nchunk = tpw // ct
    ngroup = ct // LANES
    return pl.kernel(
        _make_sc_topk_body(tpw, ct, nchunk, ngroup),
        out_type=(
            jax.ShapeDtypeStruct((TOP_K, ntok), jnp.float32),
            jax.ShapeDtypeStruct((TOP_K, ntok), jnp.int32),
        ),
        mesh=plsc.VectorSubcoreMesh(
            core_axis_name="c", subcore_axis_name="s",
            num_cores=NC, num_subcores=NS,
        ),
        scratch_types=[
            pltpu.VMEM((N_EXPERTS, ct), jnp.float32),
            pltpu.VMEM((TOP_K, ct), jnp.float32),
            pltpu.VMEM((TOP_K, ct), jnp.int32),
        ],
        compiler_params=pltpu.CompilerParams(use_tc_tiling_on_sc=False),
    )


def kernel(h, weight):
    ws, idxs = [], []
    tok0 = 0
    for ntok in SLICES:
        probs_t = _probs_transposed(h, weight, tok0, ntok)
        topk_w, topk_i = _sc_topk(ntok)(probs_t)
        ws.append(topk_w.T)
        idxs.append(topk_i.T)
        tok0 += ntok
    return jnp.concatenate(ws, axis=0), jnp.concatenate(idxs, axis=0)
